# baseline (speedup 1.0000x reference)
import functools

import jax
import jax.numpy as jnp
from jax import lax
from jax.experimental import pallas as pl
from jax.experimental.pallas import tpu as pltpu
from jax.experimental.pallas import tpu_sc as plsc

EMBED = 128
CH = 128
NBUF = 6
LANES = 16


def _pretransform(cons_table, univ_table, fusion_W, fusion_b):
    n_c, n_u = cons_table.shape[0], univ_table.shape[0]

    def body(ct, ut, fw, fb, oc, ou):
        dn = (((1,), (1,)), ((), ()))
        oc[...] = lax.dot_general(ct[...], fw[:, 1 * EMBED:2 * EMBED], dn,
                                  preferred_element_type=jnp.float32) + fb[...]
        ou[...] = lax.dot_general(ut[...], fw[:, 2 * EMBED:3 * EMBED], dn,
                                  preferred_element_type=jnp.float32)

    full = lambda r, c: pl.BlockSpec((r, c), lambda: (0, 0))
    return pl.pallas_call(
        body,
        in_specs=[full(n_c, EMBED), full(n_u, EMBED),
                  full(EMBED, 3 * EMBED), full(1, EMBED)],
        out_specs=(full(n_c, EMBED), full(n_u, EMBED)),
        out_shape=(jax.ShapeDtypeStruct((n_c, EMBED), jnp.float32),
                   jax.ShapeDtypeStruct((n_u, EMBED), jnp.float32)),
    )(cons_table, univ_table, fusion_W, fusion_b.reshape(1, EMBED))


def _sc_gather(li2, ci2, ui2, lang_tab, cons_t, univ_t):
    B = li2.size
    info = plsc.get_sparse_core_info()
    nc, ns = info.num_cores, info.num_subcores
    nw = nc * ns
    b_per_w = B // nw
    n_ch = b_per_w // CH
    mesh = plsc.VectorSubcoreMesh(core_axis_name="c", subcore_axis_name="s")

    @functools.partial(
        pl.kernel,
        mesh=mesh,
        out_type=(
            jax.ShapeDtypeStruct((B, EMBED), jnp.float32),
            jax.ShapeDtypeStruct((B, EMBED), jnp.float32),
        ),
        scratch_types=[
            pltpu.VMEM((n_ch, CH), jnp.int32),
            pltpu.VMEM((n_ch, CH), jnp.int32),
            pltpu.VMEM((n_ch, CH), jnp.int32),
            pltpu.VMEM((NBUF, CH, EMBED), jnp.float32),
            pltpu.SemaphoreType.DMA((NBUF,)),
            pltpu.SemaphoreType.DMA((NBUF,)),
            pltpu.SemaphoreType.DMA,
        ],
    )
    def k(li, ci, ui, lt, ct, ut, ol, ocomb, ixl, ixc, ixu, rows,
          sem_g, sem_w, sem_i):
        wid = lax.axis_index("s") * nc + lax.axis_index("c")
        base = wid * b_per_w
        row0 = wid * n_ch
        for src, dst in ((li, ixl), (ci, ixc), (ui, ixu)):
            pltpu.async_copy(src.at[pl.ds(row0, n_ch)], dst, sem_i).wait()

        def add_chunk(dst_ref, src_ref):
            def body(r, carry):
                for g in range(EMBED // LANES):
                    c = g * LANES
                    dst_ref[r, pl.ds(c, LANES)] = (
                        dst_ref[r, pl.ds(c, LANES)]
                        + src_ref[r, pl.ds(c, LANES)])
                return carry
            lax.fori_loop(0, CH, body, 0)

        pending_w = {}

        def writeout(s, out_slc):
            pending_w[s] = pltpu.async_copy(rows.at[s], out_slc, sem_w.at[s])

        def free_slot(s):
            h = pending_w.pop(s, None)
            if h is not None:
                h.wait()

        gh = {}

        def gather_lang(j):
            s = j % NBUF
            free_slot(s)
            gh[j] = pltpu.async_copy(lt.at[ixl.at[j]], rows.at[s],
                                     sem_g.at[s])

        for j in range(min(NBUF, n_ch)):
            gather_lang(j)
        for j in range(n_ch):
            gh[j].wait()
            writeout(j % NBUF, ol.at[pl.ds(base + j * CH, CH)])
            if j + NBUF < n_ch:
                gather_lang(j + NBUF)

        npair = NBUF // 2
        ph = {}

        def start_pair(j):
            a, b = (2 * j) % NBUF, (2 * j + 1) % NBUF
            free_slot(a)
            free_slot(b)
            ph[j] = (
                pltpu.async_copy(ct.at[ixc.at[j]], rows.at[a], sem_g.at[a]),
                pltpu.async_copy(ut.at[ixu.at[j]], rows.at[b], sem_g.at[b]),
            )

        for j in range(min(npair, n_ch)):
            start_pair(j)
        for j in range(n_ch):
            ga, gb = ph[j]
            ga.wait()
            gb.wait()
            a, b = (2 * j) % NBUF, (2 * j + 1) % NBUF
            add_chunk(rows.at[a], rows.at[b])
            writeout(a, ocomb.at[pl.ds(base + j * CH, CH)])
            if j + npair < n_ch:
                start_pair(j + npair)
        for s in list(pending_w):
            free_slot(s)

    return k(li2, ci2, ui2, lang_tab, cons_t, univ_t)


def _dense_body(le, ce, fw, g, bt, ow, ob, o):
    dn = (((1,), (1,)), ((), ()))
    x = lax.dot_general(le[...], fw[:, 0 * EMBED:1 * EMBED], dn,
                        preferred_element_type=jnp.float32)
    x += ce[...]
    mean = jnp.mean(x, axis=1, keepdims=True)
    xc = x - mean
    var = jnp.mean(xc * xc, axis=1, keepdims=True)
    xn = xc * lax.rsqrt(var + 1e-5) * g[...] + bt[...]
    o[...] = lax.dot_general(xn, ow[...], dn,
                             preferred_element_type=jnp.float32) + ob[...]


def _tc_dense(lang_e, comb_e, fusion_W, ln_gamma, ln_beta, out_W, out_b):
    B = lang_e.shape[0]
    blk = 2048
    grid = (B // blk,)
    emb_spec = pl.BlockSpec((blk, EMBED), lambda i: (i, 0))
    full = lambda r, c: pl.BlockSpec((r, c), lambda i: (0, 0))
    return pl.pallas_call(
        _dense_body,
        grid=grid,
        in_specs=[
            emb_spec, emb_spec,
            full(EMBED, 3 * EMBED),
            full(1, EMBED), full(1, EMBED),
            full(EMBED, EMBED), full(1, EMBED),
        ],
        out_specs=emb_spec,
        out_shape=jax.ShapeDtypeStruct((B, EMBED), jnp.float32),
    )(lang_e, comb_e, fusion_W,
      ln_gamma.reshape(1, EMBED), ln_beta.reshape(1, EMBED),
      out_W, out_b.reshape(1, EMBED))


def kernel(language_input, consciousness_input, universe_input, lang_table,
           cons_table, univ_table, fusion_W, fusion_b, ln_gamma, ln_beta,
           out_W, out_b):
    B = language_input.shape[0]
    li2 = language_input.astype(jnp.int32).reshape(B // CH, CH)
    ci2 = consciousness_input.astype(jnp.int32).reshape(B // CH, CH)
    ui2 = universe_input.astype(jnp.int32).reshape(B // CH, CH)
    cons_t, univ_t = _pretransform(cons_table, univ_table, fusion_W, fusion_b)
    lang_e, comb_e = _sc_gather(li2, ci2, ui2, lang_table, cons_t, univ_t)
    return _tc_dense(lang_e, comb_e, fusion_W, ln_gamma, ln_beta, out_W, out_b)


# speedup vs baseline: 1.1558x; 1.1558x over previous
import functools

import jax
import jax.numpy as jnp
from jax import lax
from jax.experimental import pallas as pl
from jax.experimental.pallas import tpu as pltpu
from jax.experimental.pallas import tpu_sc as plsc

EMBED = 128
CH = 128
NBUF = 6
LANES = 16


def _pretransform(cons_table, univ_table, fusion_W, fusion_b):
    n_c, n_u = cons_table.shape[0], univ_table.shape[0]

    def body(ct, ut, fw, fb, oc, ou):
        dn = (((1,), (1,)), ((), ()))
        oc[...] = lax.dot_general(ct[...], fw[:, 1 * EMBED:2 * EMBED], dn,
                                  preferred_element_type=jnp.float32) + fb[...]
        ou[...] = lax.dot_general(ut[...], fw[:, 2 * EMBED:3 * EMBED], dn,
                                  preferred_element_type=jnp.float32)

    full = lambda r, c: pl.BlockSpec((r, c), lambda: (0, 0))
    return pl.pallas_call(
        body,
        in_specs=[full(n_c, EMBED), full(n_u, EMBED),
                  full(EMBED, 3 * EMBED), full(1, EMBED)],
        out_specs=(full(n_c, EMBED), full(n_u, EMBED)),
        out_shape=(jax.ShapeDtypeStruct((n_c, EMBED), jnp.float32),
                   jax.ShapeDtypeStruct((n_u, EMBED), jnp.float32)),
    )(cons_table, univ_table, fusion_W, fusion_b.reshape(1, EMBED))


def _sc_gather(li2, ci2, ui2, lang_tab, cons_t, univ_t):
    B = li2.size
    info = plsc.get_sparse_core_info()
    nc, ns = info.num_cores, info.num_subcores
    nw = nc * ns
    b_per_w = B // nw
    n_ch = b_per_w // CH
    mesh = plsc.VectorSubcoreMesh(core_axis_name="c", subcore_axis_name="s")

    @functools.partial(
        pl.kernel,
        mesh=mesh,
        out_type=(
            jax.ShapeDtypeStruct((B, EMBED), jnp.float32),
            jax.ShapeDtypeStruct((B, EMBED), jnp.float32),
        ),
        scratch_types=[
            pltpu.VMEM((n_ch, CH), jnp.int32),
            pltpu.VMEM((n_ch, CH), jnp.int32),
            pltpu.VMEM((n_ch, CH), jnp.int32),
            pltpu.VMEM((NBUF, CH, EMBED), jnp.float32),
            pltpu.VMEM_SHARED((cons_t.shape[0], EMBED), jnp.float32),
            pltpu.VMEM_SHARED((univ_t.shape[0], EMBED), jnp.float32),
            pltpu.SemaphoreType.DMA((NBUF,)),
            pltpu.SemaphoreType.DMA((NBUF,)),
            pltpu.SemaphoreType.DMA,
        ],
    )
    def k(li, ci, ui, lt, ct, ut, ol, ocomb, ixl, ixc, ixu, rows,
          shc, shu, sem_g, sem_w, sem_i):
        wid = lax.axis_index("s") * nc + lax.axis_index("c")
        base = wid * b_per_w
        row0 = wid * n_ch
        for src, dst in ((li, ixl), (ci, ixc), (ui, ixu)):
            pltpu.async_copy(src.at[pl.ds(row0, n_ch)], dst, sem_i).wait()

        def add_chunk(dst_ref, src_ref):
            def body(r, carry):
                for g in range(EMBED // LANES):
                    c = g * LANES
                    dst_ref[r, pl.ds(c, LANES)] = (
                        dst_ref[r, pl.ds(c, LANES)]
                        + src_ref[r, pl.ds(c, LANES)])
                return carry
            lax.fori_loop(0, CH, body, 0)

        pending_w = {}

        def writeout(s, out_slc):
            pending_w[s] = pltpu.async_copy(rows.at[s], out_slc, sem_w.at[s])

        def free_slot(s):
            h = pending_w.pop(s, None)
            if h is not None:
                h.wait()

        gh = {}

        def gather_lang(j):
            s = j % NBUF
            free_slot(s)
            gh[j] = pltpu.async_copy(lt.at[ixl.at[j]], rows.at[s],
                                     sem_g.at[s])

        for j in range(min(NBUF, n_ch)):
            gather_lang(j)

        @pl.when(lax.axis_index("s") == 0)
        def _stage():
            pltpu.sync_copy(ct, shc)
            pltpu.sync_copy(ut, shu)

        for j in range(n_ch):
            gh[j].wait()
            writeout(j % NBUF, ol.at[pl.ds(base + j * CH, CH)])
            if j + NBUF < n_ch:
                gather_lang(j + NBUF)

        plsc.subcore_barrier()

        npair = NBUF // 2
        ph = {}

        def start_pair(j):
            a, b = (2 * j) % NBUF, (2 * j + 1) % NBUF
            free_slot(a)
            free_slot(b)
            ph[j] = (
                pltpu.async_copy(shc.at[ixc.at[j]], rows.at[a], sem_g.at[a]),
                pltpu.async_copy(shu.at[ixu.at[j]], rows.at[b], sem_g.at[b]),
            )

        for j in range(min(npair, n_ch)):
            start_pair(j)
        for j in range(n_ch):
            ga, gb = ph[j]
            ga.wait()
            gb.wait()
            a, b = (2 * j) % NBUF, (2 * j + 1) % NBUF
            add_chunk(rows.at[a], rows.at[b])
            writeout(a, ocomb.at[pl.ds(base + j * CH, CH)])
            if j + npair < n_ch:
                start_pair(j + npair)
        for s in list(pending_w):
            free_slot(s)

    return k(li2, ci2, ui2, lang_tab, cons_t, univ_t)


def _dense_body(le, ce, fw, g, bt, ow, ob, o):
    dn = (((1,), (1,)), ((), ()))
    x = lax.dot_general(le[...], fw[:, 0 * EMBED:1 * EMBED], dn,
                        preferred_element_type=jnp.float32)
    x += ce[...]
    mean = jnp.mean(x, axis=1, keepdims=True)
    xc = x - mean
    var = jnp.mean(xc * xc, axis=1, keepdims=True)
    xn = xc * lax.rsqrt(var + 1e-5) * g[...] + bt[...]
    o[...] = lax.dot_general(xn, ow[...], dn,
                             preferred_element_type=jnp.float32) + ob[...]


def _tc_dense(lang_e, comb_e, fusion_W, ln_gamma, ln_beta, out_W, out_b):
    B = lang_e.shape[0]
    blk = 4096
    grid = (B // blk,)
    emb_spec = pl.BlockSpec((blk, EMBED), lambda i: (i, 0))
    full = lambda r, c: pl.BlockSpec((r, c), lambda i: (0, 0))
    return pl.pallas_call(
        _dense_body,
        grid=grid,
        in_specs=[
            emb_spec, emb_spec,
            full(EMBED, 3 * EMBED),
            full(1, EMBED), full(1, EMBED),
            full(EMBED, EMBED), full(1, EMBED),
        ],
        out_specs=emb_spec,
        out_shape=jax.ShapeDtypeStruct((B, EMBED), jnp.float32),
    )(lang_e, comb_e, fusion_W,
      ln_gamma.reshape(1, EMBED), ln_beta.reshape(1, EMBED),
      out_W, out_b.reshape(1, EMBED))


def kernel(language_input, consciousness_input, universe_input, lang_table,
           cons_table, univ_table, fusion_W, fusion_b, ln_gamma, ln_beta,
           out_W, out_b):
    B = language_input.shape[0]
    li2 = language_input.astype(jnp.int32).reshape(B // CH, CH)
    ci2 = consciousness_input.astype(jnp.int32).reshape(B // CH, CH)
    ui2 = universe_input.astype(jnp.int32).reshape(B // CH, CH)
    cons_t, univ_t = _pretransform(cons_table, univ_table, fusion_W, fusion_b)
    lang_e, comb_e = _sc_gather(li2, ci2, ui2, lang_table, cons_t, univ_t)
    return _tc_dense(lang_e, comb_e, fusion_W, ln_gamma, ln_beta, out_W, out_b)
